# baseline (speedup 1.0000x reference)
import functools

import jax
import jax.numpy as jnp
from jax import lax
from jax.experimental import pallas as pl
from jax.experimental.pallas import tpu as pltpu

_TOP_K = 50
_MIN_KEEP = 5
_EPS = 1e-5
_PENALTY = 1.1
_TOP_P = 0.8
_CAND = 64
_NEG = float("-inf")
_INF = float("inf")
_BIGI = 2**30
_LANES = 128
_BATCH = 8


def _group_reduce2(t):
    B, TV = t.shape
    ng = TV // _LANES
    liota = lax.broadcasted_iota(jnp.int32, (B, _LANES), 1)
    giota = lax.broadcasted_iota(jnp.int32, (B, ng), 1)
    V = jnp.full((B, ng), _NEG, jnp.float32)
    NEXT = jnp.full((B, ng), _NEG, jnp.float32)
    VP = jnp.zeros((B, ng), jnp.int32)
    NP = jnp.zeros((B, ng), jnp.int32)
    for k in range(ng):
        s = t[:, k * _LANES:(k + 1) * _LANES]
        g1 = jnp.max(s, axis=1, keepdims=True)
        p1 = jnp.min(jnp.where(s == g1, liota, _BIGI), axis=1, keepdims=True)
        s2 = jnp.where(liota == p1, _NEG, s)
        g2 = jnp.max(s2, axis=1, keepdims=True)
        p2 = jnp.min(jnp.where(s2 == g2, liota, _BIGI), axis=1, keepdims=True)
        sel = giota == k
        V = jnp.where(sel, g1, V)
        VP = jnp.where(sel, p1 + k * _LANES, VP)
        NEXT = jnp.where(sel, g2, NEXT)
        NP = jnp.where(sel, p2 + k * _LANES, NP)
    return V, VP, NEXT, NP


def _round(st, base, ids, giota, tcol, ccol):
    tt, _, cv, ci = st
    V, VP, NEXT, NP = _group_reduce2(tt)
    for _j in range(_BATCH):
        vis = jnp.where(V == _INF, _NEG, V)
        winner = jnp.max(vis, axis=1, keepdims=True)
        cmin = jnp.min(cv, axis=1, keepdims=True)
        hit = winner > cmin
        tpos = jnp.min(jnp.where(vis == winner, VP, _BIGI),
                       axis=1, keepdims=True)
        ttok = base + tpos
        member = jnp.any(ids == ttok, axis=1, keepdims=True)
        pv = jnp.where(member,
                       jnp.where(winner < 0, winner * _PENALTY,
                                 winner / _PENALTY),
                       winner)
        upd = (pv > cmin) & hit
        cpos = jnp.min(jnp.where(cv == cmin, ccol, _BIGI),
                       axis=1, keepdims=True)
        sel = upd & (ccol == cpos)
        cv = jnp.where(sel, pv, cv)
        ci = jnp.where(sel, ttok, ci)
        km = hit & (giota == tpos // _LANES)
        V = jnp.where(km, NEXT, V)
        VP = jnp.where(km, NP, VP)
        NEXT = jnp.where(km, _INF, NEXT)
        tt = jnp.where(hit & (tcol == tpos), _NEG, tt)
    return tt, V, cv, ci


def _body(ids_ref, hid_ref, gamma_ref, beta_ref, w_ref,
          probs_ref, tok_ref, h_ref, cv_ref, ci_ref, l_ref, lf_ref, V):
    i = pl.program_id(0)
    nt = pl.num_programs(0)
    B = h_ref.shape[0]
    TV = w_ref.shape[0]
    ng = TV // _LANES

    @pl.when(i == 0)
    def _init():
        x = hid_ref[...]
        mu = jnp.mean(x, axis=-1, keepdims=True)
        var = jnp.var(x, axis=-1, keepdims=True)
        h = (x - mu) / jnp.sqrt(var + _EPS)
        h_ref[...] = h * gamma_ref[...] + beta_ref[...]
        cv_ref[...] = jnp.full((B, _CAND), _NEG, jnp.float32)
        ci_ref[...] = jnp.zeros((B, _CAND), jnp.int32)
        lf_ref[...] = jnp.zeros((B, 1), jnp.int32)

    t = lax.dot_general(h_ref[...], w_ref[...],
                        (((1,), (1,)), ((), ())),
                        preferred_element_type=jnp.float32)
    base = i * TV
    tcol = lax.broadcasted_iota(jnp.int32, (B, TV), 1)
    t = jnp.where(base + tcol < V, t, _NEG)

    giota = lax.broadcasted_iota(jnp.int32, (B, ng), 1)
    ccol = lax.broadcasted_iota(jnp.int32, (B, _CAND), 1)
    ids = ids_ref[...]
    rnd = functools.partial(_round, base=base, ids=ids, giota=giota,
                            tcol=tcol, ccol=ccol)
    v0 = jnp.full((B, ng), _INF, jnp.float32)

    l_ref[i] = t
    nr = jnp.where(i < 2, 10, jnp.where(i < 5, 6, jnp.where(i < 16, 4, 2)))

    def round_body(_r, carry):
        vv, cv, ci = carry
        tt, vv, cv, ci = rnd((l_ref[i], vv, cv, ci))
        l_ref[i] = tt
        return vv, cv, ci

    vv, cv, ci = lax.fori_loop(0, nr, round_body,
                               (v0, cv_ref[...], ci_ref[...]))
    cv_ref[...] = cv
    ci_ref[...] = ci
    left = jnp.max(vv, axis=1, keepdims=True) > \
        jnp.min(cv, axis=1, keepdims=True)
    lf_ref[...] = lf_ref[...] | left.astype(jnp.int32)

    @pl.when(i == nt - 1)
    def _finalize():
        @pl.when(jnp.any(lf_ref[...] > 0))
        def _slow():
            def tile_fix(j, carry):
                cv, ci = carry

                def wcond(st):
                    tt, vv, cv, ci = st
                    return jnp.any(jnp.max(tt, axis=1) >
                                   jnp.min(cv, axis=1))

                def wbody(st):
                    return _round(st, base=j * TV, ids=ids, giota=giota,
                                  tcol=tcol, ccol=ccol)

                tt0 = l_ref[j]
                _, _, cv, ci = lax.while_loop(
                    wcond, wbody, (tt0, v0, cv, ci))
                return cv, ci

            cv, ci = lax.fori_loop(0, nt, tile_fix,
                                   (cv_ref[...], ci_ref[...]))
            cv_ref[...] = cv
            ci_ref[...] = ci

        cv = cv_ref[...]
        ci = ci_ref[...]
        sv = jnp.full((B, _CAND), _NEG, jnp.float32)
        stok = jnp.zeros((B, _CAND), jnp.int32)
        for r in range(_TOP_K):
            m = jnp.max(cv, axis=1, keepdims=True)
            mtok = jnp.min(jnp.where(cv == m, ci, _BIGI), axis=1, keepdims=True)
            sv = jnp.where(ccol == r, m, sv)
            stok = jnp.where(ccol == r, mtok, stok)
            cv = jnp.where((cv == m) & (ci == mtok), _NEG, cv)
        mx = jnp.max(sv, axis=1, keepdims=True)
        ex = jnp.exp(sv - mx)
        p = ex / jnp.sum(ex, axis=1, keepdims=True)
        tri = (lax.broadcasted_iota(jnp.int32, (_CAND, _CAND), 0)
               <= lax.broadcasted_iota(jnp.int32, (_CAND, _CAND), 1)
               ).astype(jnp.float32)
        cum = lax.dot_general(p, tri, (((1,), (0,)), ((), ())),
                              precision=lax.Precision.HIGHEST,
                              preferred_element_type=jnp.float32)
        keepm = (cum < _TOP_P) | (ccol < _MIN_KEEP)
        filt = jnp.where(keepm, sv, jnp.float32(-1000.0))
        fmx = jnp.max(filt, axis=1, keepdims=True)
        fex = jnp.exp(filt - fmx)
        probs = fex / jnp.sum(fex, axis=1, keepdims=True)
        probs_ref[...] = probs[:, :_TOP_K]
        tok_ref[...] = stok[:, :_TOP_K]


def kernel(input_ids, hidden_states, gamma, beta, W):
    B, D = hidden_states.shape
    V = W.shape[0]
    HIST = input_ids.shape[1]
    TV = 2048
    nt = -(-V // TV)

    in_specs = [
        pl.BlockSpec((B, HIST), lambda i: (0, 0)),
        pl.BlockSpec((B, D), lambda i: (0, 0)),
        pl.BlockSpec((1, D), lambda i: (0, 0)),
        pl.BlockSpec((1, D), lambda i: (0, 0)),
        pl.BlockSpec((TV, D), lambda i: (i, 0)),
    ]
    out_specs = [
        pl.BlockSpec((B, _TOP_K), lambda i: (0, 0)),
        pl.BlockSpec((B, _TOP_K), lambda i: (0, 0)),
    ]
    probs, token = pl.pallas_call(
        functools.partial(_body, V=V),
        grid=(nt,),
        in_specs=in_specs,
        out_specs=out_specs,
        out_shape=[
            jax.ShapeDtypeStruct((B, _TOP_K), jnp.float32),
            jax.ShapeDtypeStruct((B, _TOP_K), jnp.int32),
        ],
        scratch_shapes=[
            pltpu.VMEM((B, D), jnp.float32),
            pltpu.VMEM((B, _CAND), jnp.float32),
            pltpu.VMEM((B, _CAND), jnp.int32),
            pltpu.VMEM((nt, B, TV), jnp.float32),
            pltpu.VMEM((B, 1), jnp.int32),
        ],
        compiler_params=pltpu.CompilerParams(
            dimension_semantics=("arbitrary",)),
    )(input_ids, hidden_states, gamma.reshape(1, D), beta.reshape(1, D), W)
    return probs, token


# speedup vs baseline: 1.3024x; 1.3024x over previous
import functools

import jax
import jax.numpy as jnp
from jax import lax
from jax.experimental import pallas as pl
from jax.experimental.pallas import tpu as pltpu

_TOP_K = 50
_MIN_KEEP = 5
_EPS = 1e-5
_PENALTY = 1.1
_TOP_P = 0.8
_CAND = 64
_NEG = float("-inf")
_INF = float("inf")
_BIGI = 2**30
_LANES = 128
_BATCH = 8
_EARLY = 8


def _group_reduce2(t):
    B, TV = t.shape
    ng = TV // _LANES
    liota = lax.broadcasted_iota(jnp.int32, (B, _LANES), 1)
    giota = lax.broadcasted_iota(jnp.int32, (B, ng), 1)
    V = jnp.full((B, ng), _NEG, jnp.float32)
    NEXT = jnp.full((B, ng), _NEG, jnp.float32)
    VP = jnp.zeros((B, ng), jnp.int32)
    NP = jnp.zeros((B, ng), jnp.int32)
    for k in range(ng):
        s = t[:, k * _LANES:(k + 1) * _LANES]
        g1 = jnp.max(s, axis=1, keepdims=True)
        p1 = jnp.min(jnp.where(s == g1, liota, _BIGI), axis=1, keepdims=True)
        s2 = jnp.where(liota == p1, _NEG, s)
        g2 = jnp.max(s2, axis=1, keepdims=True)
        p2 = jnp.min(jnp.where(s2 == g2, liota, _BIGI), axis=1, keepdims=True)
        sel = giota == k
        V = jnp.where(sel, g1, V)
        VP = jnp.where(sel, p1 + k * _LANES, VP)
        NEXT = jnp.where(sel, g2, NEXT)
        NP = jnp.where(sel, p2 + k * _LANES, NP)
    return V, VP, NEXT, NP


def _round(st, base, ids, giota, tcol, ccol):
    tt, _, cv, ci = st
    V, VP, NEXT, NP = _group_reduce2(tt)
    for _j in range(_BATCH):
        vis = jnp.where(V == _INF, _NEG, V)
        winner = jnp.max(vis, axis=1, keepdims=True)
        cmin = jnp.min(cv, axis=1, keepdims=True)
        hit = winner > cmin
        tpos = jnp.min(jnp.where(vis == winner, VP, _BIGI),
                       axis=1, keepdims=True)
        ttok = base + tpos
        member = jnp.any(ids == ttok, axis=1, keepdims=True)
        pv = jnp.where(member,
                       jnp.where(winner < 0, winner * _PENALTY,
                                 winner / _PENALTY),
                       winner)
        upd = (pv > cmin) & hit
        cpos = jnp.min(jnp.where(cv == cmin, ccol, _BIGI),
                       axis=1, keepdims=True)
        sel = upd & (ccol == cpos)
        cv = jnp.where(sel, pv, cv)
        ci = jnp.where(sel, ttok, ci)
        km = hit & (giota == tpos // _LANES)
        V = jnp.where(km, NEXT, V)
        VP = jnp.where(km, NP, VP)
        NEXT = jnp.where(km, _INF, NEXT)
        tt = jnp.where(hit & (tcol == tpos), _NEG, tt)
    return tt, V, cv, ci


def _body(ids_ref, hid_ref, gamma_ref, beta_ref, w_ref,
          probs_ref, tok_ref, h_ref, cv_ref, ci_ref, l_ref, lf_ref, V):
    i = pl.program_id(0)
    nt = pl.num_programs(0)
    B = h_ref.shape[0]
    TV = w_ref.shape[0]
    ng = TV // _LANES

    @pl.when(i == 0)
    def _init():
        x = hid_ref[...]
        mu = jnp.mean(x, axis=-1, keepdims=True)
        var = jnp.var(x, axis=-1, keepdims=True)
        h = (x - mu) / jnp.sqrt(var + _EPS)
        h_ref[...] = h * gamma_ref[...] + beta_ref[...]
        cv_ref[...] = jnp.full((B, _CAND), _NEG, jnp.float32)
        ci_ref[...] = jnp.zeros((B, _CAND), jnp.int32)
        lf_ref[...] = jnp.zeros((B, 1), jnp.int32)

    t = lax.dot_general(h_ref[...], w_ref[...],
                        (((1,), (1,)), ((), ())),
                        preferred_element_type=jnp.float32)
    base = i * TV
    tcol = lax.broadcasted_iota(jnp.int32, (B, TV), 1)
    t = jnp.where(base + tcol < V, t, _NEG)

    giota = lax.broadcasted_iota(jnp.int32, (B, ng), 1)
    ccol = lax.broadcasted_iota(jnp.int32, (B, _CAND), 1)
    ids = ids_ref[...]
    rnd = functools.partial(_round, base=base, ids=ids, giota=giota,
                            tcol=tcol, ccol=ccol)
    v0 = jnp.full((B, ng), _INF, jnp.float32)

    st = (t, v0, cv_ref[...], ci_ref[...])
    st = rnd(st)
    tt, vv, cv, ci = rnd(st)
    l_ref[i] = tt
    cv_ref[...] = cv
    ci_ref[...] = ci
    left = (jnp.max(vv, axis=1, keepdims=True) >
            jnp.min(cv, axis=1, keepdims=True)) & (i >= _EARLY)
    lf_ref[...] = lf_ref[...] | left.astype(jnp.int32)

    @pl.when(i == nt - 1)
    def _finalize():
        def tile_fix(j, carry):
            cv, ci = carry

            def wcond(st):
                tt, vv, cv, ci = st
                return jnp.any(jnp.max(tt, axis=1) > jnp.min(cv, axis=1))

            def wbody(st):
                return _round(st, base=j * TV, ids=ids, giota=giota,
                              tcol=tcol, ccol=ccol)

            tt0 = l_ref[j]
            _, _, cv, ci = lax.while_loop(wcond, wbody, (tt0, v0, cv, ci))
            return cv, ci

        cv, ci = lax.fori_loop(0, min(_EARLY, nt), tile_fix,
                               (cv_ref[...], ci_ref[...]))
        cv_ref[...] = cv
        ci_ref[...] = ci

        if nt > _EARLY:
            @pl.when(jnp.any(lf_ref[...] > 0))
            def _slow():
                cv, ci = lax.fori_loop(_EARLY, nt, tile_fix,
                                       (cv_ref[...], ci_ref[...]))
                cv_ref[...] = cv
                ci_ref[...] = ci

        cv = cv_ref[...]
        ci = ci_ref[...]
        sv = jnp.full((B, _CAND), _NEG, jnp.float32)
        stok = jnp.zeros((B, _CAND), jnp.int32)
        for r in range(_TOP_K):
            m = jnp.max(cv, axis=1, keepdims=True)
            mtok = jnp.min(jnp.where(cv == m, ci, _BIGI), axis=1, keepdims=True)
            sv = jnp.where(ccol == r, m, sv)
            stok = jnp.where(ccol == r, mtok, stok)
            cv = jnp.where((cv == m) & (ci == mtok), _NEG, cv)
        mx = jnp.max(sv, axis=1, keepdims=True)
        ex = jnp.exp(sv - mx)
        p = ex / jnp.sum(ex, axis=1, keepdims=True)
        tri = (lax.broadcasted_iota(jnp.int32, (_CAND, _CAND), 0)
               <= lax.broadcasted_iota(jnp.int32, (_CAND, _CAND), 1)
               ).astype(jnp.float32)
        cum = lax.dot_general(p, tri, (((1,), (0,)), ((), ())),
                              precision=lax.Precision.HIGHEST,
                              preferred_element_type=jnp.float32)
        keepm = (cum < _TOP_P) | (ccol < _MIN_KEEP)
        filt = jnp.where(keepm, sv, jnp.float32(-1000.0))
        fmx = jnp.max(filt, axis=1, keepdims=True)
        fex = jnp.exp(filt - fmx)
        probs = fex / jnp.sum(fex, axis=1, keepdims=True)
        probs_ref[...] = probs[:, :_TOP_K]
        tok_ref[...] = stok[:, :_TOP_K]


def kernel(input_ids, hidden_states, gamma, beta, W):
    B, D = hidden_states.shape
    V = W.shape[0]
    HIST = input_ids.shape[1]
    TV = 2048
    nt = -(-V // TV)

    in_specs = [
        pl.BlockSpec((B, HIST), lambda i: (0, 0)),
        pl.BlockSpec((B, D), lambda i: (0, 0)),
        pl.BlockSpec((1, D), lambda i: (0, 0)),
        pl.BlockSpec((1, D), lambda i: (0, 0)),
        pl.BlockSpec((TV, D), lambda i: (i, 0)),
    ]
    out_specs = [
        pl.BlockSpec((B, _TOP_K), lambda i: (0, 0)),
        pl.BlockSpec((B, _TOP_K), lambda i: (0, 0)),
    ]
    probs, token = pl.pallas_call(
        functools.partial(_body, V=V),
        grid=(nt,),
        in_specs=in_specs,
        out_specs=out_specs,
        out_shape=[
            jax.ShapeDtypeStruct((B, _TOP_K), jnp.float32),
            jax.ShapeDtypeStruct((B, _TOP_K), jnp.int32),
        ],
        scratch_shapes=[
            pltpu.VMEM((B, D), jnp.float32),
            pltpu.VMEM((B, _CAND), jnp.float32),
            pltpu.VMEM((B, _CAND), jnp.int32),
            pltpu.VMEM((nt, B, TV), jnp.float32),
            pltpu.VMEM((B, 1), jnp.int32),
        ],
        compiler_params=pltpu.CompilerParams(
            dimension_semantics=("arbitrary",)),
    )(input_ids, hidden_states, gamma.reshape(1, D), beta.reshape(1, D), W)
    return probs, token


# speedup vs baseline: 1.3071x; 1.0036x over previous
import functools

import jax
import jax.numpy as jnp
from jax import lax
from jax.experimental import pallas as pl
from jax.experimental.pallas import tpu as pltpu

_TOP_K = 50
_MIN_KEEP = 5
_EPS = 1e-5
_PENALTY = 1.1
_TOP_P = 0.8
_CAND = 64
_NEG = float("-inf")
_INF = float("inf")
_BIGI = 2**30
_LANES = 128
_BATCH = 8
_EARLY = 8


def _group_reduce2(t):
    B, TV = t.shape
    ng = TV // _LANES
    liota = lax.broadcasted_iota(jnp.int32, (B, _LANES), 1)
    giota = lax.broadcasted_iota(jnp.int32, (B, ng), 1)
    V = jnp.full((B, ng), _NEG, jnp.float32)
    NEXT = jnp.full((B, ng), _NEG, jnp.float32)
    VP = jnp.zeros((B, ng), jnp.int32)
    NP = jnp.zeros((B, ng), jnp.int32)
    for k in range(ng):
        s = t[:, k * _LANES:(k + 1) * _LANES]
        g1 = jnp.max(s, axis=1, keepdims=True)
        p1 = jnp.min(jnp.where(s == g1, liota, _BIGI), axis=1, keepdims=True)
        s2 = jnp.where(liota == p1, _NEG, s)
        g2 = jnp.max(s2, axis=1, keepdims=True)
        p2 = jnp.min(jnp.where(s2 == g2, liota, _BIGI), axis=1, keepdims=True)
        sel = giota == k
        V = jnp.where(sel, g1, V)
        VP = jnp.where(sel, p1 + k * _LANES, VP)
        NEXT = jnp.where(sel, g2, NEXT)
        NP = jnp.where(sel, p2 + k * _LANES, NP)
    return V, VP, NEXT, NP


def _round(st, base, ids, giota, tcol, ccol):
    tt, _, cv, ci = st
    V, VP, NEXT, NP = _group_reduce2(tt)
    for _j in range(_BATCH):
        vis = jnp.where(V == _INF, _NEG, V)
        winner = jnp.max(vis, axis=1, keepdims=True)
        cmin = jnp.min(cv, axis=1, keepdims=True)
        hit = winner > cmin
        tpos = jnp.min(jnp.where(vis == winner, VP, _BIGI),
                       axis=1, keepdims=True)
        ttok = base + tpos
        member = jnp.any(ids == ttok, axis=1, keepdims=True)
        pv = jnp.where(member,
                       jnp.where(winner < 0, winner * _PENALTY,
                                 winner / _PENALTY),
                       winner)
        upd = (pv > cmin) & hit
        cpos = jnp.min(jnp.where(cv == cmin, ccol, _BIGI),
                       axis=1, keepdims=True)
        sel = upd & (ccol == cpos)
        cv = jnp.where(sel, pv, cv)
        ci = jnp.where(sel, ttok, ci)
        km = hit & (giota == tpos // _LANES)
        V = jnp.where(km, NEXT, V)
        VP = jnp.where(km, NP, VP)
        NEXT = jnp.where(km, _INF, NEXT)
        tt = jnp.where(hit & (tcol == tpos), _NEG, tt)
    return tt, V, cv, ci


def _body(ids_ref, hid_ref, gamma_ref, beta_ref, w_ref,
          probs_ref, tok_ref, h_ref, cv_ref, ci_ref, l_ref, lf_ref, V):
    i = pl.program_id(0)
    nt = pl.num_programs(0)
    B = h_ref.shape[0]
    TV = w_ref.shape[0]
    ng = TV // _LANES

    @pl.when(i == 0)
    def _init():
        x = hid_ref[...]
        mu = jnp.mean(x, axis=-1, keepdims=True)
        var = jnp.var(x, axis=-1, keepdims=True)
        h = (x - mu) / jnp.sqrt(var + _EPS)
        h_ref[...] = h * gamma_ref[...] + beta_ref[...]
        cv_ref[...] = jnp.full((B, _CAND), _NEG, jnp.float32)
        ci_ref[...] = jnp.zeros((B, _CAND), jnp.int32)
        lf_ref[...] = jnp.zeros((B, 1), jnp.int32)

    t = lax.dot_general(h_ref[...], w_ref[...],
                        (((1,), (1,)), ((), ())),
                        preferred_element_type=jnp.float32)
    base = i * TV
    tcol = lax.broadcasted_iota(jnp.int32, (B, TV), 1)
    t = jnp.where(base + tcol < V, t, _NEG)

    giota = lax.broadcasted_iota(jnp.int32, (B, ng), 1)
    ccol = lax.broadcasted_iota(jnp.int32, (B, _CAND), 1)
    ids = ids_ref[...]
    rnd = functools.partial(_round, base=base, ids=ids, giota=giota,
                            tcol=tcol, ccol=ccol)
    v0 = jnp.full((B, ng), _INF, jnp.float32)

    st = (t, v0, cv_ref[...], ci_ref[...])
    st = rnd(st)
    tt, vv, cv, ci = rnd(st)
    l_ref[i] = tt
    cv_ref[...] = cv
    ci_ref[...] = ci
    left = (jnp.max(vv, axis=1, keepdims=True) >
            jnp.min(cv, axis=1, keepdims=True)) & (i >= _EARLY)
    lf_ref[...] = lf_ref[...] | left.astype(jnp.int32)

    @pl.when(i == nt - 1)
    def _finalize():
        def tile_fix(j, carry):
            cv, ci = carry

            def wcond(st):
                tt, vv, cv, ci = st
                return jnp.any(jnp.max(tt, axis=1) > jnp.min(cv, axis=1))

            def wbody(st):
                return _round(st, base=j * TV, ids=ids, giota=giota,
                              tcol=tcol, ccol=ccol)

            tt0 = l_ref[j]
            _, _, cv, ci = lax.while_loop(wcond, wbody, (tt0, v0, cv, ci))
            return cv, ci

        ew = min(_EARLY, nt)
        ecat = jnp.concatenate([l_ref[j] for j in range(ew)], axis=1)
        egio = lax.broadcasted_iota(jnp.int32, (B, ew * ng), 1)
        etcol = lax.broadcasted_iota(jnp.int32, (B, ew * TV), 1)
        ev0 = jnp.full((B, ew * ng), _INF, jnp.float32)

        def ewcond(st):
            tt, vv, cv, ci = st
            return jnp.any(jnp.max(tt, axis=1) > jnp.min(cv, axis=1))

        def ewbody(st):
            return _round(st, base=0, ids=ids, giota=egio,
                          tcol=etcol, ccol=ccol)

        _, _, cv, ci = lax.while_loop(
            ewcond, ewbody, (ecat, ev0, cv_ref[...], ci_ref[...]))
        cv_ref[...] = cv
        ci_ref[...] = ci

        if nt > _EARLY:
            @pl.when(jnp.any(lf_ref[...] > 0))
            def _slow():
                cv, ci = lax.fori_loop(_EARLY, nt, tile_fix,
                                       (cv_ref[...], ci_ref[...]))
                cv_ref[...] = cv
                ci_ref[...] = ci

        cv = cv_ref[...]
        ci = ci_ref[...]
        sv = jnp.full((B, _CAND), _NEG, jnp.float32)
        stok = jnp.zeros((B, _CAND), jnp.int32)
        for r in range(_TOP_K):
            m = jnp.max(cv, axis=1, keepdims=True)
            mtok = jnp.min(jnp.where(cv == m, ci, _BIGI), axis=1, keepdims=True)
            sv = jnp.where(ccol == r, m, sv)
            stok = jnp.where(ccol == r, mtok, stok)
            cv = jnp.where((cv == m) & (ci == mtok), _NEG, cv)
        mx = jnp.max(sv, axis=1, keepdims=True)
        ex = jnp.exp(sv - mx)
        p = ex / jnp.sum(ex, axis=1, keepdims=True)
        tri = (lax.broadcasted_iota(jnp.int32, (_CAND, _CAND), 0)
               <= lax.broadcasted_iota(jnp.int32, (_CAND, _CAND), 1)
               ).astype(jnp.float32)
        cum = lax.dot_general(p, tri, (((1,), (0,)), ((), ())),
                              precision=lax.Precision.HIGHEST,
                              preferred_element_type=jnp.float32)
        keepm = (cum < _TOP_P) | (ccol < _MIN_KEEP)
        filt = jnp.where(keepm, sv, jnp.float32(-1000.0))
        fmx = jnp.max(filt, axis=1, keepdims=True)
        fex = jnp.exp(filt - fmx)
        probs = fex / jnp.sum(fex, axis=1, keepdims=True)
        probs_ref[...] = probs[:, :_TOP_K]
        tok_ref[...] = stok[:, :_TOP_K]


def kernel(input_ids, hidden_states, gamma, beta, W):
    B, D = hidden_states.shape
    V = W.shape[0]
    HIST = input_ids.shape[1]
    TV = 2048
    nt = -(-V // TV)

    in_specs = [
        pl.BlockSpec((B, HIST), lambda i: (0, 0)),
        pl.BlockSpec((B, D), lambda i: (0, 0)),
        pl.BlockSpec((1, D), lambda i: (0, 0)),
        pl.BlockSpec((1, D), lambda i: (0, 0)),
        pl.BlockSpec((TV, D), lambda i: (i, 0)),
    ]
    out_specs = [
        pl.BlockSpec((B, _TOP_K), lambda i: (0, 0)),
        pl.BlockSpec((B, _TOP_K), lambda i: (0, 0)),
    ]
    probs, token = pl.pallas_call(
        functools.partial(_body, V=V),
        grid=(nt,),
        in_specs=in_specs,
        out_specs=out_specs,
        out_shape=[
            jax.ShapeDtypeStruct((B, _TOP_K), jnp.float32),
            jax.ShapeDtypeStruct((B, _TOP_K), jnp.int32),
        ],
        scratch_shapes=[
            pltpu.VMEM((B, D), jnp.float32),
            pltpu.VMEM((B, _CAND), jnp.float32),
            pltpu.VMEM((B, _CAND), jnp.int32),
            pltpu.VMEM((nt, B, TV), jnp.float32),
            pltpu.VMEM((B, 1), jnp.int32),
        ],
        compiler_params=pltpu.CompilerParams(
            dimension_semantics=("arbitrary",)),
    )(input_ids, hidden_states, gamma.reshape(1, D), beta.reshape(1, D), W)
    return probs, token
